# baseline (speedup 1.0000x reference)
import functools
import math

import jax
import jax.numpy as jnp
import numpy as np
from jax.experimental import pallas as pl

_NUM_EXPERTS = 64
_K = 8
_DIM = 768
_FREQ_DIM = 256
_B = 128
_HW = 16
_S = _HW * _HW
_NOISE_STD = 1.0 / _NUM_EXPERTS
_TAU = 1.0
_NEG = -1e30
_PB = 8
_NSTEPS = _B // _PB


def _fused_body(xt_ref, freq_ref, wg_ref, wf_ref, comp_ref, noise_ref,
                gates_ref, idx_ref, vals_ref, aux_ref, pooled_ref):
    f32 = jnp.float32
    i = pl.program_id(0)
    pooled_ref[pl.ds(i * _PB, _PB), :] = (
        jnp.sum(xt_ref[...], axis=1) * (1.0 / _S))

    @pl.when(i == _NSTEPS - 1)
    def _router():
        pooled = pooled_ref[...]
        logits = jax.lax.dot_general(
            pooled, wg_ref[...], (((1,), (1,)), ((), ())),
            precision=jax.lax.Precision.DEFAULT, preferred_element_type=f32)
        logits = logits + jax.lax.dot_general(
            freq_ref[...], wf_ref[...], (((1,), (1,)), ((), ())),
            precision=jax.lax.Precision.DEFAULT, preferred_element_type=f32)

        m = jnp.max(logits, axis=-1, keepdims=True)
        e = jnp.exp(logits - m)
        clean = e / jnp.sum(e, axis=-1, keepdims=True)
        importance = (jnp.sum(clean, axis=0, keepdims=True)
                      * comp_ref[...] * _TAU)
        imp_mean = (jnp.sum(importance, axis=1, keepdims=True)
                    * (1.0 / _NUM_EXPERTS))
        imp_var = jnp.sum((importance - imp_mean) ** 2, axis=1,
                          keepdims=True) * (1.0 / (_NUM_EXPERTS - 1))
        loss_imp = imp_var / (imp_mean + 1e-8) ** 2

        noisy = logits + noise_ref[...]
        m2 = jnp.max(noisy, axis=-1, keepdims=True)
        e2 = jnp.exp(noisy - m2)
        gprobs = e2 / jnp.sum(e2, axis=-1, keepdims=True)

        iota = jax.lax.broadcasted_iota(jnp.int32, (_B, _NUM_EXPERTS), 1)
        work = noisy
        gates = jnp.zeros((_B, _NUM_EXPERTS), f32)
        thr = None
        for k in range(_K):
            mk = jnp.max(work, axis=-1, keepdims=True)
            idxk = jnp.min(jnp.where(work == mk, iota, _NUM_EXPERTS),
                           axis=-1, keepdims=True)
            onehot = iota == idxk
            valk = jnp.sum(jnp.where(onehot, gprobs, 0.0), axis=-1,
                           keepdims=True)
            gates = jnp.where(onehot, gprobs, gates)
            idx_ref[:, k:k + 1] = idxk
            vals_ref[:, k:k + 1] = valk
            work = jnp.where(onehot, _NEG, work)
            if k == _K - 1:
                thr = mk
        gates_ref[...] = gates

        inv_sqrt2 = 1.0 / math.sqrt(2.0)
        nr = (thr - logits) * (1.0 / _NOISE_STD)
        p = 1.0 - 0.5 * (1.0 + jax.lax.erf(nr * inv_sqrt2))
        p_mean = jnp.sum(p, axis=0, keepdims=True) * (1.0 / _B)
        pmm = jnp.sum(p_mean, axis=1, keepdims=True) * (1.0 / _NUM_EXPERTS)
        p_var = jnp.sum((p_mean - pmm) ** 2, axis=1, keepdims=True) * (
            1.0 / (_NUM_EXPERTS - 1))
        loss_load = p_var / (pmm + 1e-8) ** 2

        aux_ref[...] = 0.5 * loss_imp + 0.5 * loss_load


@functools.partial(jax.jit, static_argnames=("interpret",))
def _impl(x, freq_emb, W_gate, W_freq, complexity, interpret=False):
    from jax.experimental.pallas import tpu as pltpu

    noise = jax.random.normal(
        jax.random.key(1), (_B, _NUM_EXPERTS), dtype=jnp.float32) * _NOISE_STD
    xt = jnp.transpose(x, (0, 2, 3, 1)).reshape(_B, _S, _DIM)
    comp2 = complexity.reshape(1, _NUM_EXPERTS)

    small = [freq_emb, W_gate, W_freq, comp2, noise]
    gates, idx, vals, aux = pl.pallas_call(
        _fused_body,
        grid=(_NSTEPS,),
        in_specs=[pl.BlockSpec((_PB, _S, _DIM), lambda i: (i, 0, 0))] + [
            pl.BlockSpec(a.shape, functools.partial(
                lambda nd, i: (0,) * nd, a.ndim)) for a in small],
        out_specs=(
            pl.BlockSpec((_B, _NUM_EXPERTS), lambda i: (0, 0)),
            pl.BlockSpec((_B, _K), lambda i: (0, 0)),
            pl.BlockSpec((_B, _K), lambda i: (0, 0)),
            pl.BlockSpec((1, 1), lambda i: (0, 0)),
        ),
        out_shape=(
            jax.ShapeDtypeStruct((_B, _NUM_EXPERTS), jnp.float32),
            jax.ShapeDtypeStruct((_B, _K), jnp.int32),
            jax.ShapeDtypeStruct((_B, _K), jnp.float32),
            jax.ShapeDtypeStruct((1, 1), jnp.float32),
        ),
        scratch_shapes=[pltpu.VMEM((_B, _DIM), jnp.float32)],
        interpret=interpret,
    )(xt, *small)
    return gates, idx, vals, aux[0, 0]


def kernel(x, freq_emb, W_gate, W_freq, complexity):
    return _impl(x, freq_emb, W_gate, W_freq, complexity)


# speedup vs baseline: 1.4746x; 1.4746x over previous
import functools
import math

import jax
import jax.numpy as jnp
import numpy as np
from jax.experimental import pallas as pl

_NUM_EXPERTS = 64
_K = 8
_DIM = 768
_FREQ_DIM = 256
_B = 128
_HW = 16
_S = _HW * _HW
_NOISE_STD = 1.0 / _NUM_EXPERTS
_TAU = 1.0
_NEG = -1e30
_PB = 8
_NSTEPS = _B // _PB


def _fused_body(xt_ref, xt2_ref, freq_ref, wg_ref, wf_ref, comp_ref, noise_ref,
                gates_ref, idx_ref, vals_ref, aux_ref, pooled_ref):
    f32 = jnp.float32
    i = pl.program_id(0)
    pooled_ref[pl.ds(i * _PB, _PB), :] = (
        jnp.sum(xt_ref[...], axis=1) * (1.0 / _S))
    pooled_ref[pl.ds((_NSTEPS // 2 + i) * _PB, _PB), :] = (
        jnp.sum(xt2_ref[...], axis=1) * (1.0 / _S))

    @pl.when(i == _NSTEPS // 2 - 1)
    def _router():
        pooled = pooled_ref[...]
        logits = jax.lax.dot_general(
            pooled, wg_ref[...], (((1,), (1,)), ((), ())),
            precision=jax.lax.Precision.DEFAULT, preferred_element_type=f32)
        logits = logits + jax.lax.dot_general(
            freq_ref[...], wf_ref[...], (((1,), (1,)), ((), ())),
            precision=jax.lax.Precision.DEFAULT, preferred_element_type=f32)

        m = jnp.max(logits, axis=-1, keepdims=True)
        e = jnp.exp(logits - m)
        clean = e / jnp.sum(e, axis=-1, keepdims=True)
        importance = (jnp.sum(clean, axis=0, keepdims=True)
                      * comp_ref[...] * _TAU)
        imp_mean = (jnp.sum(importance, axis=1, keepdims=True)
                    * (1.0 / _NUM_EXPERTS))
        imp_var = jnp.sum((importance - imp_mean) ** 2, axis=1,
                          keepdims=True) * (1.0 / (_NUM_EXPERTS - 1))
        loss_imp = imp_var / (imp_mean + 1e-8) ** 2

        noisy = logits + noise_ref[...]
        m2 = jnp.max(noisy, axis=-1, keepdims=True)
        e2 = jnp.exp(noisy - m2)
        gprobs = e2 / jnp.sum(e2, axis=-1, keepdims=True)

        iota = jax.lax.broadcasted_iota(jnp.int32, (_B, _NUM_EXPERTS), 1)
        work = noisy
        gates = jnp.zeros((_B, _NUM_EXPERTS), f32)
        thr = None
        for k in range(_K):
            mk = jnp.max(work, axis=-1, keepdims=True)
            idxk = jnp.min(jnp.where(work == mk, iota, _NUM_EXPERTS),
                           axis=-1, keepdims=True)
            onehot = iota == idxk
            valk = jnp.sum(jnp.where(onehot, gprobs, 0.0), axis=-1,
                           keepdims=True)
            gates = jnp.where(onehot, gprobs, gates)
            idx_ref[:, k:k + 1] = idxk
            vals_ref[:, k:k + 1] = valk
            work = jnp.where(onehot, _NEG, work)
            if k == _K - 1:
                thr = mk
        gates_ref[...] = gates

        inv_sqrt2 = 1.0 / math.sqrt(2.0)
        nr = (thr - logits) * (1.0 / _NOISE_STD)
        p = 1.0 - 0.5 * (1.0 + jax.lax.erf(nr * inv_sqrt2))
        p_mean = jnp.sum(p, axis=0, keepdims=True) * (1.0 / _B)
        pmm = jnp.sum(p_mean, axis=1, keepdims=True) * (1.0 / _NUM_EXPERTS)
        p_var = jnp.sum((p_mean - pmm) ** 2, axis=1, keepdims=True) * (
            1.0 / (_NUM_EXPERTS - 1))
        loss_load = p_var / (pmm + 1e-8) ** 2

        aux_ref[...] = 0.5 * loss_imp + 0.5 * loss_load


@functools.partial(jax.jit, static_argnames=("interpret",))
def _impl(x, freq_emb, W_gate, W_freq, complexity, interpret=False):
    from jax.experimental.pallas import tpu as pltpu

    noise = jax.random.normal(
        jax.random.key(1), (_B, _NUM_EXPERTS), dtype=jnp.float32) * _NOISE_STD
    xt = jnp.transpose(x, (0, 2, 3, 1)).reshape(_B, _S, _DIM)
    comp2 = complexity.reshape(1, _NUM_EXPERTS)

    small = [freq_emb, W_gate, W_freq, comp2, noise]
    gates, idx, vals, aux = pl.pallas_call(
        _fused_body,
        grid=(_NSTEPS // 2,),
        in_specs=[pl.BlockSpec((_PB, _S, _DIM), lambda i: (i, 0, 0)),
                  pl.BlockSpec((_PB, _S, _DIM),
                               lambda i: (i + _NSTEPS // 2, 0, 0))] + [
            pl.BlockSpec(a.shape, functools.partial(
                lambda nd, i: (0,) * nd, a.ndim)) for a in small],
        out_specs=(
            pl.BlockSpec((_B, _NUM_EXPERTS), lambda i: (0, 0)),
            pl.BlockSpec((_B, _K), lambda i: (0, 0)),
            pl.BlockSpec((_B, _K), lambda i: (0, 0)),
            pl.BlockSpec((1, 1), lambda i: (0, 0)),
        ),
        out_shape=(
            jax.ShapeDtypeStruct((_B, _NUM_EXPERTS), jnp.float32),
            jax.ShapeDtypeStruct((_B, _K), jnp.int32),
            jax.ShapeDtypeStruct((_B, _K), jnp.float32),
            jax.ShapeDtypeStruct((1, 1), jnp.float32),
        ),
        scratch_shapes=[pltpu.VMEM((_B, _DIM), jnp.float32)],
        interpret=interpret,
    )(xt, xt, *small)
    return gates, idx, vals, aux[0, 0]


def kernel(x, freq_emb, W_gate, W_freq, complexity):
    return _impl(x, freq_emb, W_gate, W_freq, complexity)


# speedup vs baseline: 1.5111x; 1.0247x over previous
import functools
import math

import jax
import jax.numpy as jnp
from jax.experimental import pallas as pl

_NUM_EXPERTS = 64
_K = 8
_DIM = 768
_FREQ_DIM = 256
_B = 128
_HW = 16
_S = _HW * _HW
_NOISE_STD = 1.0 / _NUM_EXPERTS
_TAU = 1.0
_NEG = -1e30
_PB = 8
_NSTEPS = _B // _PB


def _fused_body(xt_ref, freq_ref, wg_ref, wf_ref, comp_ref, noise_ref,
                gates_ref, idx_ref, vals_ref, aux_ref, pooled_ref):
    f32 = jnp.float32
    i = pl.program_id(0)
    pooled_ref[pl.ds(i * _PB, _PB), :] = (
        jnp.sum(xt_ref[...], axis=1) * (1.0 / _S))

    @pl.when(i == _NSTEPS - 1)
    def _router():
        pooled = pooled_ref[...]
        logits = jax.lax.dot_general(
            pooled, wg_ref[...], (((1,), (1,)), ((), ())),
            precision=jax.lax.Precision.DEFAULT, preferred_element_type=f32)
        logits = logits + jax.lax.dot_general(
            freq_ref[...], wf_ref[...], (((1,), (1,)), ((), ())),
            precision=jax.lax.Precision.DEFAULT, preferred_element_type=f32)

        m = jnp.max(logits, axis=-1, keepdims=True)
        e = jnp.exp(logits - m)
        clean = e / jnp.sum(e, axis=-1, keepdims=True)
        importance = (jnp.sum(clean, axis=0, keepdims=True)
                      * comp_ref[...] * _TAU)
        imp_mean = (jnp.sum(importance, axis=1, keepdims=True)
                    * (1.0 / _NUM_EXPERTS))
        imp_var = jnp.sum((importance - imp_mean) ** 2, axis=1,
                          keepdims=True) * (1.0 / (_NUM_EXPERTS - 1))
        loss_imp = imp_var / (imp_mean + 1e-8) ** 2

        noisy = logits + noise_ref[...]
        m2 = jnp.max(noisy, axis=-1, keepdims=True)
        e2 = jnp.exp(noisy - m2)
        gprobs = e2 / jnp.sum(e2, axis=-1, keepdims=True)

        iota = jax.lax.broadcasted_iota(jnp.int32, (_B, _NUM_EXPERTS), 1)
        work = noisy
        gates = jnp.zeros((_B, _NUM_EXPERTS), f32)
        thr = None
        for k in range(_K):
            mk = jnp.max(work, axis=-1, keepdims=True)
            idxk = jnp.min(jnp.where(work == mk, iota, _NUM_EXPERTS),
                           axis=-1, keepdims=True)
            onehot = iota == idxk
            valk = jnp.sum(jnp.where(onehot, gprobs, 0.0), axis=-1,
                           keepdims=True)
            gates = jnp.where(onehot, gprobs, gates)
            idx_ref[:, k:k + 1] = idxk
            vals_ref[:, k:k + 1] = valk
            work = jnp.where(onehot, _NEG, work)
            if k == _K - 1:
                thr = mk
        gates_ref[...] = gates

        inv_sqrt2 = 1.0 / math.sqrt(2.0)
        nr = (thr - logits) * (1.0 / _NOISE_STD)
        p = 1.0 - 0.5 * (1.0 + jax.lax.erf(nr * inv_sqrt2))
        p_mean = jnp.sum(p, axis=0, keepdims=True) * (1.0 / _B)
        pmm = jnp.sum(p_mean, axis=1, keepdims=True) * (1.0 / _NUM_EXPERTS)
        p_var = jnp.sum((p_mean - pmm) ** 2, axis=1, keepdims=True) * (
            1.0 / (_NUM_EXPERTS - 1))
        loss_load = p_var / (pmm + 1e-8) ** 2

        aux_ref[...] = 0.5 * loss_imp + 0.5 * loss_load


@jax.jit
def _impl(x, freq_emb, W_gate, W_freq, complexity):
    from jax.experimental.pallas import tpu as pltpu

    noise = jax.random.normal(
        jax.random.key(1), (_B, _NUM_EXPERTS), dtype=jnp.float32) * _NOISE_STD
    xt = jnp.transpose(x, (0, 2, 3, 1)).reshape(_B, _S, _DIM)
    comp2 = complexity.reshape(1, _NUM_EXPERTS)

    small = [freq_emb, W_gate, W_freq, comp2, noise]
    gates, idx, vals, aux = pl.pallas_call(
        _fused_body,
        grid=(_NSTEPS,),
        in_specs=[pl.BlockSpec((_PB, _S, _DIM), lambda i: (i, 0, 0))] + [
            pl.BlockSpec(a.shape, functools.partial(
                lambda nd, i: (0,) * nd, a.ndim)) for a in small],
        out_specs=(
            pl.BlockSpec((_B, _NUM_EXPERTS), lambda i: (0, 0)),
            pl.BlockSpec((_B, _K), lambda i: (0, 0)),
            pl.BlockSpec((_B, _K), lambda i: (0, 0)),
            pl.BlockSpec((1, 1), lambda i: (0, 0)),
        ),
        out_shape=(
            jax.ShapeDtypeStruct((_B, _NUM_EXPERTS), jnp.float32),
            jax.ShapeDtypeStruct((_B, _K), jnp.int32),
            jax.ShapeDtypeStruct((_B, _K), jnp.float32),
            jax.ShapeDtypeStruct((1, 1), jnp.float32),
        ),
        scratch_shapes=[pltpu.VMEM((_B, _DIM), jnp.float32)],
    )(xt, *small)
    return gates, idx, vals, aux[0, 0]


def kernel(x, freq_emb, W_gate, W_freq, complexity):
    return _impl(x, freq_emb, W_gate, W_freq, complexity)
